# Initial kernel scaffold; baseline (speedup 1.0000x reference)
import jax
import jax.numpy as jnp
from jax.experimental import pallas as pl


def kernel(weights, atom_types, system_ids, n_systems):
    raise NotImplementedError("write your pallas kernel here")


# speedup vs baseline: 55.2237x; 55.2237x over previous
import functools

import jax
import jax.numpy as jnp
from jax import lax
from jax.experimental import pallas as pl
from jax.experimental.pallas import tpu as pltpu
from jax.experimental.pallas import tpu_sc as plsc

N_ATOMS = 1048576
N_TYPES = 119
N_PROPS = 64
N_SYSTEMS = 16384

STRIDE = 120
NBINS = N_SYSTEMS * STRIDE
NC = 2
NS = 16
NW = NC * NS
A_PER_W = N_ATOMS // NW
CHUNK = 2048
ACC_PER_TILE = NBINS // NS


def _sc_histogram(types3d, sys3d, zeros_h, ones_h):
    mesh = plsc.VectorSubcoreMesh(
        core_axis_name="c", subcore_axis_name="s", num_cores=NC, num_subcores=NS
    )

    @functools.partial(
        pl.kernel,
        mesh=mesh,
        out_type=(
            jax.ShapeDtypeStruct((NBINS,), jnp.float32),
            jax.ShapeDtypeStruct((NBINS,), jnp.float32),
        ),
        scratch_types=[
            pltpu.VMEM((CHUNK,), jnp.int32),
            pltpu.VMEM((CHUNK,), jnp.int32),
            pltpu.VMEM((CHUNK,), jnp.float32),
            pltpu.VMEM_SHARED((NBINS,), jnp.float32),
        ],
    )
    def hist(types_hbm, sys_hbm, zeros_hbm, ones_hbm, out0, out1,
             keys_v, types_v, ones_v, acc):
        c = lax.axis_index("c")
        s = lax.axis_index("s")
        w = c * NS + s

        pltpu.sync_copy(ones_hbm, ones_v)
        pltpu.sync_copy(zeros_hbm, acc.at[pl.ds(s * ACC_PER_TILE, ACC_PER_TILE)])
        plsc.subcore_barrier()

        def chunk(k, carry):
            base = k * CHUNK
            pltpu.sync_copy(sys_hbm.at[w, pl.ds(base, CHUNK)], keys_v)
            pltpu.sync_copy(types_hbm.at[w, pl.ds(base, CHUNK)], types_v)

            def vec(r, carry2):
                sl = pl.ds(r * 16, 16)
                keys_v[sl] = keys_v[sl] * STRIDE + types_v[sl]
                return carry2

            lax.fori_loop(0, CHUNK // 16, vec, 0)
            pltpu.sync_copy(ones_v, acc.at[keys_v], add=True)
            return carry

        lax.fori_loop(0, A_PER_W // CHUNK, chunk, 0)
        plsc.subcore_barrier()

        sl = pl.ds(s * ACC_PER_TILE, ACC_PER_TILE)

        @pl.when(c == 0)
        def _():
            pltpu.sync_copy(acc.at[sl], out0.at[sl])

        @pl.when(c == 1)
        def _():
            pltpu.sync_copy(acc.at[sl], out1.at[sl])

    return hist(types3d, sys3d, zeros_h, ones_h)


_BM = 2048


def _matmul_body(c0_ref, c1_ref, w_ref, o_ref):
    counts = c0_ref[...] + c1_ref[...]
    o_ref[...] = lax.dot_general(
        counts, w_ref[...], (((1,), (0,)), ((), ())),
        preferred_element_type=jnp.float32,
    )


def _tc_matmul(c0, c1, w_pad):
    return pl.pallas_call(
        _matmul_body,
        grid=(N_SYSTEMS // _BM,),
        in_specs=[
            pl.BlockSpec((_BM, STRIDE), lambda i: (i, 0)),
            pl.BlockSpec((_BM, STRIDE), lambda i: (i, 0)),
            pl.BlockSpec((STRIDE, N_PROPS), lambda i: (0, 0)),
        ],
        out_specs=pl.BlockSpec((_BM, N_PROPS), lambda i: (i, 0)),
        out_shape=jax.ShapeDtypeStruct((N_SYSTEMS, N_PROPS), jnp.float32),
    )(c0, c1, w_pad)


def kernel(weights, atom_types, system_ids, n_systems):
    del n_systems
    types3d = atom_types.reshape(NW, A_PER_W)
    sys3d = system_ids.reshape(NW, A_PER_W)
    zeros_h = jnp.zeros((ACC_PER_TILE,), jnp.float32)
    ones_h = jnp.ones((CHUNK,), jnp.float32)
    c0, c1 = _sc_histogram(types3d, sys3d, zeros_h, ones_h)
    w_pad = jnp.zeros((STRIDE, N_PROPS), jnp.float32).at[:N_TYPES].set(weights)
    return _tc_matmul(c0.reshape(N_SYSTEMS, STRIDE),
                      c1.reshape(N_SYSTEMS, STRIDE), w_pad)


# speedup vs baseline: 55.8678x; 1.0117x over previous
import functools

import jax
import jax.numpy as jnp
from jax import lax
from jax.experimental import pallas as pl
from jax.experimental.pallas import tpu as pltpu
from jax.experimental.pallas import tpu_sc as plsc

N_ATOMS = 1048576
N_TYPES = 119
N_PROPS = 64
N_SYSTEMS = 16384

TPAD = 120
NBINS = TPAD * N_SYSTEMS
NC = 2
NS = 16
NW = NC * NS
A_PER_W = N_ATOMS // NW
CHUNK = 2048
ACC_PER_TILE = NBINS // NS


def _sc_histogram(atom_types, system_ids, zeros_h, ones_h):
    mesh = plsc.VectorSubcoreMesh(
        core_axis_name="c", subcore_axis_name="s", num_cores=NC, num_subcores=NS
    )

    @functools.partial(
        pl.kernel,
        mesh=mesh,
        out_type=(
            jax.ShapeDtypeStruct((NBINS,), jnp.float32),
            jax.ShapeDtypeStruct((NBINS,), jnp.float32),
        ),
        scratch_types=[
            pltpu.VMEM((CHUNK,), jnp.int32),
            pltpu.VMEM((CHUNK,), jnp.int32),
            pltpu.VMEM((CHUNK,), jnp.float32),
            pltpu.VMEM_SHARED((NBINS,), jnp.float32),
        ],
    )
    def hist(types_hbm, sys_hbm, zeros_hbm, ones_hbm, out0, out1,
             keys_v, sys_v, ones_v, acc):
        c = lax.axis_index("c")
        s = lax.axis_index("s")
        w = c * NS + s

        pltpu.sync_copy(ones_hbm, ones_v)
        pltpu.sync_copy(zeros_hbm, acc.at[pl.ds(s * ACC_PER_TILE, ACC_PER_TILE)])
        plsc.subcore_barrier()

        def chunk(k, carry):
            base = w * A_PER_W + k * CHUNK
            pltpu.sync_copy(types_hbm.at[pl.ds(base, CHUNK)], keys_v)
            pltpu.sync_copy(sys_hbm.at[pl.ds(base, CHUNK)], sys_v)

            def vec(r, carry2):
                sl = pl.ds(r * 16, 16)
                keys_v[sl] = keys_v[sl] * N_SYSTEMS + sys_v[sl]
                return carry2

            lax.fori_loop(0, CHUNK // 16, vec, 0)
            pltpu.sync_copy(ones_v, acc.at[keys_v], add=True)
            return carry

        lax.fori_loop(0, A_PER_W // CHUNK, chunk, 0)
        plsc.subcore_barrier()

        sl = pl.ds(s * ACC_PER_TILE, ACC_PER_TILE)

        @pl.when(c == 0)
        def _():
            pltpu.sync_copy(acc.at[sl], out0.at[sl])

        @pl.when(c == 1)
        def _():
            pltpu.sync_copy(acc.at[sl], out1.at[sl])

    return hist(atom_types, system_ids, zeros_h, ones_h)


_BM = 2048


def _matmul_body(c0_ref, c1_ref, w_ref, o_ref):
    counts = c0_ref[...] + c1_ref[...]
    o_ref[...] = lax.dot_general(
        counts, w_ref[...], (((0,), (0,)), ((), ())),
        preferred_element_type=jnp.float32,
    )


def _tc_matmul(c0, c1, w_pad):
    return pl.pallas_call(
        _matmul_body,
        grid=(N_SYSTEMS // _BM,),
        in_specs=[
            pl.BlockSpec((TPAD, _BM), lambda i: (0, i)),
            pl.BlockSpec((TPAD, _BM), lambda i: (0, i)),
            pl.BlockSpec((TPAD, N_PROPS), lambda i: (0, 0)),
        ],
        out_specs=pl.BlockSpec((_BM, N_PROPS), lambda i: (i, 0)),
        out_shape=jax.ShapeDtypeStruct((N_SYSTEMS, N_PROPS), jnp.float32),
    )(c0, c1, w_pad)


def kernel(weights, atom_types, system_ids, n_systems):
    del n_systems
    zeros_h = jnp.zeros((ACC_PER_TILE,), jnp.float32)
    ones_h = jnp.ones((CHUNK,), jnp.float32)
    c0, c1 = _sc_histogram(atom_types, system_ids, zeros_h, ones_h)
    w_pad = jnp.zeros((TPAD, N_PROPS), jnp.float32).at[:N_TYPES].set(weights)
    return _tc_matmul(c0.reshape(TPAD, N_SYSTEMS),
                      c1.reshape(TPAD, N_SYSTEMS), w_pad)


# speedup vs baseline: 65.9479x; 1.1804x over previous
import functools

import jax
import jax.numpy as jnp
from jax import lax
from jax.experimental import pallas as pl
from jax.experimental.pallas import tpu as pltpu
from jax.experimental.pallas import tpu_sc as plsc

N_ATOMS = 1048576
N_TYPES = 119
N_PROPS = 64
N_SYSTEMS = 16384

TPAD = 120
NBINS = TPAD * N_SYSTEMS
NC = 2
NS = 16
NW = NC * NS
A_PER_W = N_ATOMS // NW
CHUNK = 1024
NCH = A_PER_W // CHUNK
ACC_PER_TILE = NBINS // NS


def _sc_histogram(atom_types, system_ids, zeros_h, ones_h):
    mesh = plsc.VectorSubcoreMesh(
        core_axis_name="c", subcore_axis_name="s", num_cores=NC, num_subcores=NS
    )

    @functools.partial(
        pl.kernel,
        mesh=mesh,
        out_type=(
            jax.ShapeDtypeStruct((NBINS,), jnp.float32),
            jax.ShapeDtypeStruct((NBINS,), jnp.float32),
        ),
        scratch_types=[
            pltpu.VMEM((CHUNK,), jnp.int32),
            pltpu.VMEM((CHUNK,), jnp.int32),
            pltpu.VMEM((CHUNK,), jnp.int32),
            pltpu.VMEM((CHUNK,), jnp.int32),
            pltpu.VMEM((CHUNK,), jnp.float32),
            pltpu.VMEM_SHARED((NBINS,), jnp.float32),
            pltpu.SemaphoreType.DMA,
            pltpu.SemaphoreType.DMA,
            pltpu.SemaphoreType.DMA,
            pltpu.SemaphoreType.DMA,
            pltpu.SemaphoreType.DMA,
        ],
    )
    def hist(types_hbm, sys_hbm, zeros_hbm, ones_hbm, out0, out1,
             t0_v, t1_v, s0_v, s1_v, ones_v,
             acc, sem_z, sem_l0, sem_l1, sem_s0, sem_s1):
        c = lax.axis_index("c")
        s = lax.axis_index("s")
        w = c * NS + s
        t_buf, s_buf = (t0_v, t1_v), (s0_v, s1_v)
        sem_l, sem_s = (sem_l0, sem_l1), (sem_s0, sem_s1)

        zd = pltpu.async_copy(
            zeros_hbm, acc.at[pl.ds(s * ACC_PER_TILE, ACC_PER_TILE)], sem_z)
        pltpu.sync_copy(ones_hbm, ones_v)

        def start_loads(k, b):
            base = w * A_PER_W + k * CHUNK
            lt = pltpu.async_copy(types_hbm.at[pl.ds(base, CHUNK)],
                                  t_buf[b], sem_l[b])
            ls = pltpu.async_copy(sys_hbm.at[pl.ds(base, CHUNK)],
                                  s_buf[b], sem_l[b])
            return lt, ls

        loads = start_loads(0, 0)
        zd.wait()
        plsc.subcore_barrier()

        scatters = [None, None]
        for k in range(NCH):
            b = k % 2
            loads[0].wait()
            loads[1].wait()

            kb, sb = t_buf[b], s_buf[b]

            def vec(r, carry, kb=kb, sb=sb):
                sl = pl.ds(r * 16, 16)
                kb[sl] = kb[sl] * N_SYSTEMS + sb[sl]
                return carry

            lax.fori_loop(0, CHUNK // 16, vec, 0)

            if k + 1 < NCH:
                if scatters[1 - b] is not None:
                    scatters[1 - b].wait()
                    scatters[1 - b] = None
                loads = start_loads(k + 1, 1 - b)

            scatters[b] = pltpu.async_copy(
                ones_v, acc.at[t_buf[b]], sem_s[b], add=True)

        for d in scatters:
            if d is not None:
                d.wait()
        plsc.subcore_barrier()

        sl = pl.ds(s * ACC_PER_TILE, ACC_PER_TILE)

        @pl.when(c == 0)
        def _():
            pltpu.sync_copy(acc.at[sl], out0.at[sl])

        @pl.when(c == 1)
        def _():
            pltpu.sync_copy(acc.at[sl], out1.at[sl])

    return hist(atom_types, system_ids, zeros_h, ones_h)


_BM = 2048


def _matmul_body(c_ref, w_ref, o_ref):
    o_ref[...] = lax.dot_general(
        c_ref[...], w_ref[...], (((0,), (0,)), ((), ())),
        preferred_element_type=jnp.float32,
    )


def _tc_matmul(cnt, w_pad):
    return pl.pallas_call(
        _matmul_body,
        grid=(N_SYSTEMS // _BM,),
        in_specs=[
            pl.BlockSpec((TPAD, _BM), lambda i: (0, i)),
            pl.BlockSpec((TPAD, N_PROPS), lambda i: (0, 0)),
        ],
        out_specs=pl.BlockSpec((_BM, N_PROPS), lambda i: (i, 0)),
        out_shape=jax.ShapeDtypeStruct((N_SYSTEMS, N_PROPS), jnp.float32),
    )(cnt, w_pad)


def kernel(weights, atom_types, system_ids, n_systems):
    del n_systems
    zeros_h = jnp.zeros((ACC_PER_TILE,), jnp.float32)
    ones_h = jnp.ones((CHUNK,), jnp.float32)
    c0, c1 = _sc_histogram(atom_types, system_ids, zeros_h, ones_h)
    cnt = (c0 + c1).reshape(TPAD, N_SYSTEMS)
    w_pad = jnp.zeros((TPAD, N_PROPS), jnp.float32).at[:N_TYPES].set(weights)
    return _tc_matmul(cnt, w_pad)


# speedup vs baseline: 67.7295x; 1.0270x over previous
import functools

import jax
import jax.numpy as jnp
from jax import lax
from jax.experimental import pallas as pl
from jax.experimental.pallas import tpu as pltpu
from jax.experimental.pallas import tpu_sc as plsc

N_ATOMS = 1048576
N_TYPES = 119
N_PROPS = 64
N_SYSTEMS = 16384

TPAD = 120
NBINS = TPAD * N_SYSTEMS
NC = 2
NS = 16
NW = NC * NS
A_PER_W = N_ATOMS // NW
CHUNK = 1024
NCH = A_PER_W // CHUNK
ACC_PER_TILE = NBINS // NS


def _sc_histogram(atom_types, system_ids, zeros_h, ones_h):
    mesh = plsc.VectorSubcoreMesh(
        core_axis_name="c", subcore_axis_name="s", num_cores=NC, num_subcores=NS
    )

    @functools.partial(
        pl.kernel,
        mesh=mesh,
        out_type=(
            jax.ShapeDtypeStruct((NBINS,), jnp.float32),
            jax.ShapeDtypeStruct((NBINS,), jnp.float32),
        ),
        scratch_types=[
            pltpu.VMEM((CHUNK,), jnp.int32),
            pltpu.VMEM((CHUNK,), jnp.int32),
            pltpu.VMEM((CHUNK,), jnp.int32),
            pltpu.VMEM((CHUNK,), jnp.int32),
            pltpu.VMEM((CHUNK,), jnp.float32),
            pltpu.VMEM_SHARED((NBINS,), jnp.float32),
            pltpu.SemaphoreType.DMA,
            pltpu.SemaphoreType.DMA,
            pltpu.SemaphoreType.DMA,
            pltpu.SemaphoreType.DMA,
            pltpu.SemaphoreType.DMA,
        ],
    )
    def hist(types_hbm, sys_hbm, zeros_hbm, ones_hbm, out0, out1,
             t0_v, t1_v, s0_v, s1_v, ones_v,
             acc, sem_z, sem_l0, sem_l1, sem_s0, sem_s1):
        c = lax.axis_index("c")
        s = lax.axis_index("s")
        w = c * NS + s
        t_buf, s_buf = (t0_v, t1_v), (s0_v, s1_v)
        sem_l, sem_s = (sem_l0, sem_l1), (sem_s0, sem_s1)

        zd = pltpu.async_copy(
            zeros_hbm, acc.at[pl.ds(s * ACC_PER_TILE, ACC_PER_TILE)], sem_z)
        pltpu.sync_copy(ones_hbm, ones_v)

        def start_loads(k, b):
            base = w * A_PER_W + k * CHUNK
            lt = pltpu.async_copy(types_hbm.at[pl.ds(base, CHUNK)],
                                  t_buf[b], sem_l[b])
            ls = pltpu.async_copy(sys_hbm.at[pl.ds(base, CHUNK)],
                                  s_buf[b], sem_l[b])
            return lt, ls

        loads = start_loads(0, 0)
        zd.wait()
        plsc.subcore_barrier()

        scatters = [None, None]
        for k in range(NCH):
            b = k % 2
            loads[0].wait()
            loads[1].wait()

            kb, sb = t_buf[b], s_buf[b]

            @plsc.parallel_loop(0, CHUNK, step=16, unroll=8)
            def vec(r, kb=kb, sb=sb):
                sl = pl.ds(r, 16)
                kb[sl] = kb[sl] * N_SYSTEMS + sb[sl]

            if k + 1 < NCH:
                if scatters[1 - b] is not None:
                    scatters[1 - b].wait()
                    scatters[1 - b] = None
                loads = start_loads(k + 1, 1 - b)

            scatters[b] = pltpu.async_copy(
                ones_v, acc.at[t_buf[b]], sem_s[b], add=True)

        for d in scatters:
            if d is not None:
                d.wait()
        plsc.subcore_barrier()

        sl = pl.ds(s * ACC_PER_TILE, ACC_PER_TILE)

        @pl.when(c == 0)
        def _():
            pltpu.sync_copy(acc.at[sl], out0.at[sl])

        @pl.when(c == 1)
        def _():
            pltpu.sync_copy(acc.at[sl], out1.at[sl])

    return hist(atom_types, system_ids, zeros_h, ones_h)


_BM = 2048


def _matmul_body(c_ref, w_ref, o_ref):
    o_ref[...] = lax.dot_general(
        c_ref[...], w_ref[...], (((0,), (0,)), ((), ())),
        preferred_element_type=jnp.float32,
    )


def _tc_matmul(cnt, w_pad):
    return pl.pallas_call(
        _matmul_body,
        grid=(N_SYSTEMS // _BM,),
        in_specs=[
            pl.BlockSpec((TPAD, _BM), lambda i: (0, i)),
            pl.BlockSpec((TPAD, N_PROPS), lambda i: (0, 0)),
        ],
        out_specs=pl.BlockSpec((_BM, N_PROPS), lambda i: (i, 0)),
        out_shape=jax.ShapeDtypeStruct((N_SYSTEMS, N_PROPS), jnp.float32),
    )(cnt, w_pad)


def kernel(weights, atom_types, system_ids, n_systems):
    del n_systems
    zeros_h = jnp.zeros((ACC_PER_TILE,), jnp.float32)
    ones_h = jnp.ones((CHUNK,), jnp.float32)
    c0, c1 = _sc_histogram(atom_types, system_ids, zeros_h, ones_h)
    cnt = (c0 + c1).reshape(TPAD, N_SYSTEMS)
    w_pad = jnp.zeros((TPAD, N_PROPS), jnp.float32).at[:N_TYPES].set(weights)
    return _tc_matmul(cnt, w_pad)


# speedup vs baseline: 75.6517x; 1.1170x over previous
import functools

import jax
import jax.numpy as jnp
from jax import lax
from jax.experimental import pallas as pl
from jax.experimental.pallas import tpu as pltpu
from jax.experimental.pallas import tpu_sc as plsc

N_ATOMS = 1048576
N_TYPES = 119
N_PROPS = 64
N_SYSTEMS = 16384

TPAD = 120
NBINS = TPAD * N_SYSTEMS
NC = 2
NS = 16
NW = NC * NS
A_PER_W = N_ATOMS // NW
CHUNK = 1024
NBUF = 6
NCH = A_PER_W // CHUNK
ACC_PER_TILE = NBINS // NS


def _sc_histogram(keys, zeros_h, ones_h):
    mesh = plsc.VectorSubcoreMesh(
        core_axis_name="c", subcore_axis_name="s", num_cores=NC, num_subcores=NS
    )

    @functools.partial(
        pl.kernel,
        mesh=mesh,
        out_type=(
            jax.ShapeDtypeStruct((NBINS,), jnp.float32),
            jax.ShapeDtypeStruct((NBINS,), jnp.float32),
        ),
        scratch_types=[
            [pltpu.VMEM((CHUNK,), jnp.int32) for _ in range(NBUF)],
            pltpu.VMEM((CHUNK,), jnp.float32),
            pltpu.VMEM_SHARED((NBINS,), jnp.float32),
            pltpu.SemaphoreType.DMA,
            [pltpu.SemaphoreType.DMA for _ in range(NBUF)],
            [pltpu.SemaphoreType.DMA for _ in range(NBUF)],
        ],
    )
    def hist(keys_hbm, zeros_hbm, ones_hbm, out0, out1,
             kbufs, ones_v, acc, sem_z, sem_l, sem_s):
        c = lax.axis_index("c")
        s = lax.axis_index("s")
        w = c * NS + s

        zd = pltpu.async_copy(
            zeros_hbm, acc.at[pl.ds(s * ACC_PER_TILE, ACC_PER_TILE)], sem_z)
        pltpu.sync_copy(ones_hbm, ones_v)

        def start_load(k):
            b = k % NBUF
            base = w * A_PER_W + k * CHUNK
            return pltpu.async_copy(keys_hbm.at[pl.ds(base, CHUNK)],
                                    kbufs[b], sem_l[b])

        loads = [start_load(k) for k in range(NBUF - 1)] + [None] * (
            NCH - (NBUF - 1))
        zd.wait()
        plsc.subcore_barrier()

        scatters = [None] * NBUF
        for k in range(NCH):
            b = k % NBUF
            loads[k].wait()
            scatters[b] = pltpu.async_copy(
                ones_v, acc.at[kbufs[b]], sem_s[b], add=True)
            nxt = k + NBUF - 1
            if nxt < NCH:
                nb = nxt % NBUF
                if scatters[nb] is not None:
                    scatters[nb].wait()
                    scatters[nb] = None
                loads[nxt] = start_load(nxt)

        for d in scatters:
            if d is not None:
                d.wait()
        plsc.subcore_barrier()

        sl = pl.ds(s * ACC_PER_TILE, ACC_PER_TILE)

        @pl.when(c == 0)
        def _():
            pltpu.sync_copy(acc.at[sl], out0.at[sl])

        @pl.when(c == 1)
        def _():
            pltpu.sync_copy(acc.at[sl], out1.at[sl])

    return hist(keys, zeros_h, ones_h)


_BM = 4096


def _matmul_body(c_ref, w_ref, o_ref):
    o_ref[...] = lax.dot_general(
        c_ref[...], w_ref[...], (((0,), (0,)), ((), ())),
        preferred_element_type=jnp.float32,
    )


def _tc_matmul(cnt, w_pad):
    return pl.pallas_call(
        _matmul_body,
        grid=(N_SYSTEMS // _BM,),
        in_specs=[
            pl.BlockSpec((TPAD, _BM), lambda i: (0, i)),
            pl.BlockSpec((TPAD, N_PROPS), lambda i: (0, 0)),
        ],
        out_specs=pl.BlockSpec((_BM, N_PROPS), lambda i: (i, 0)),
        out_shape=jax.ShapeDtypeStruct((N_SYSTEMS, N_PROPS), jnp.float32),
    )(cnt, w_pad)


def kernel(weights, atom_types, system_ids, n_systems):
    del n_systems
    keys = atom_types * N_SYSTEMS + system_ids
    zeros_h = jnp.zeros((ACC_PER_TILE,), jnp.float32)
    ones_h = jnp.ones((CHUNK,), jnp.float32)
    c0, c1 = _sc_histogram(keys, zeros_h, ones_h)
    cnt = (c0 + c1).reshape(TPAD, N_SYSTEMS)
    w_pad = jnp.zeros((TPAD, N_PROPS), jnp.float32).at[:N_TYPES].set(weights)
    return _tc_matmul(cnt, w_pad)


# speedup vs baseline: 85.2189x; 1.1265x over previous
import functools

import jax
import jax.numpy as jnp
from jax import lax
from jax.experimental import pallas as pl
from jax.experimental.pallas import tpu as pltpu
from jax.experimental.pallas import tpu_sc as plsc

N_ATOMS = 1048576
N_TYPES = 119
N_PROPS = 64
N_SYSTEMS = 16384

TPAD = 120
NBINS = TPAD * N_SYSTEMS
NC = 2
NS = 16
NW = NC * NS
A_PER_W = N_ATOMS // NW
CHUNK = 1024
NBUF = 6
NCH = A_PER_W // CHUNK
ACC_PER_TILE = NBINS // NS


def _sc_histogram(keys, zeros_h, ones_h):
    mesh = plsc.VectorSubcoreMesh(
        core_axis_name="c", subcore_axis_name="s", num_cores=NC, num_subcores=NS
    )

    @functools.partial(
        pl.kernel,
        mesh=mesh,
        out_type=(
            jax.ShapeDtypeStruct((TPAD, N_SYSTEMS), jnp.float32),
            jax.ShapeDtypeStruct((TPAD, N_SYSTEMS), jnp.float32),
        ),
        scratch_types=[
            [pltpu.VMEM((CHUNK,), jnp.int32) for _ in range(NBUF)],
            pltpu.VMEM((CHUNK,), jnp.float32),
            pltpu.VMEM_SHARED((NBINS,), jnp.float32),
            pltpu.SemaphoreType.DMA,
            [pltpu.SemaphoreType.DMA for _ in range(NBUF)],
            [pltpu.SemaphoreType.DMA for _ in range(NBUF)],
        ],
    )
    def hist(keys_hbm, zeros_hbm, ones_hbm, out0, out1,
             kbufs, ones_v, acc, sem_z, sem_l, sem_s):
        c = lax.axis_index("c")
        s = lax.axis_index("s")
        w = c * NS + s

        zd = pltpu.async_copy(
            zeros_hbm, acc.at[pl.ds(s * ACC_PER_TILE, ACC_PER_TILE)], sem_z)
        pltpu.sync_copy(ones_hbm, ones_v)

        def start_load(k):
            b = k % NBUF
            base = w * A_PER_W + k * CHUNK
            return pltpu.async_copy(keys_hbm.at[pl.ds(base, CHUNK)],
                                    kbufs[b], sem_l[b])

        loads = [start_load(k) for k in range(NBUF - 1)] + [None] * (
            NCH - (NBUF - 1))
        zd.wait()
        plsc.subcore_barrier()

        scatters = [None] * NBUF
        for k in range(NCH):
            b = k % NBUF
            loads[k].wait()
            scatters[b] = pltpu.async_copy(
                ones_v, acc.at[kbufs[b]], sem_s[b], add=True)
            nxt = k + NBUF - 1
            if nxt < NCH:
                nb = nxt % NBUF
                if scatters[nb] is not None:
                    scatters[nb].wait()
                    scatters[nb] = None
                loads[nxt] = start_load(nxt)

        for d in scatters:
            if d is not None:
                d.wait()
        plsc.subcore_barrier()

        for i in range(8):
            r = i * NS + s

            @pl.when(r < TPAD)
            def _(r=r):
                row = acc.at[pl.ds(r * N_SYSTEMS, N_SYSTEMS)]

                @pl.when(c == 0)
                def _():
                    pltpu.sync_copy(row, out0.at[r])

                @pl.when(c == 1)
                def _():
                    pltpu.sync_copy(row, out1.at[r])

    return hist(keys, zeros_h, ones_h)


_BN = 2048
_NB = N_SYSTEMS // _BN


def _fused_body(c0_hbm, c1_hbm, w_ref, o_ref, lhs0, lhs1, sems):
    def start(j, b):
        sl = pl.ds(j * _BN, _BN)
        d0 = pltpu.make_async_copy(c0_hbm.at[:, sl], lhs0.at[b], sems.at[b, 0])
        d1 = pltpu.make_async_copy(c1_hbm.at[:, sl], lhs1.at[b], sems.at[b, 1])
        d0.start()
        d1.start()
        return d0, d1

    pend = start(0, 0)
    for j in range(_NB):
        b = j % 2
        pend[0].wait()
        pend[1].wait()
        if j + 1 < _NB:
            pend = start(j + 1, 1 - b)
        cnt = lhs0[b] + lhs1[b]
        o_ref[pl.ds(j * _BN, _BN), :] = lax.dot_general(
            cnt, w_ref[...], (((0,), (0,)), ((), ())),
            preferred_element_type=jnp.float32,
        )


def _tc_matmul(c0, c1, w_pad):
    return pl.pallas_call(
        _fused_body,
        in_specs=[
            pl.BlockSpec(memory_space=pltpu.HBM),
            pl.BlockSpec(memory_space=pltpu.HBM),
            pl.BlockSpec(memory_space=pltpu.VMEM),
        ],
        out_specs=pl.BlockSpec(memory_space=pltpu.VMEM),
        out_shape=jax.ShapeDtypeStruct((N_SYSTEMS, N_PROPS), jnp.float32),
        scratch_shapes=[
            pltpu.VMEM((2, TPAD, _BN), jnp.float32),
            pltpu.VMEM((2, TPAD, _BN), jnp.float32),
            pltpu.SemaphoreType.DMA((2, 2)),
        ],
    )(c0, c1, w_pad)


def kernel(weights, atom_types, system_ids, n_systems):
    del n_systems
    keys = atom_types * N_SYSTEMS + system_ids
    zeros_h = jnp.zeros((ACC_PER_TILE,), jnp.float32)
    ones_h = jnp.ones((CHUNK,), jnp.float32)
    c0, c1 = _sc_histogram(keys, zeros_h, ones_h)
    w_pad = jnp.zeros((TPAD, N_PROPS), jnp.float32).at[:N_TYPES].set(weights)
    return _tc_matmul(c0, c1, w_pad)


# speedup vs baseline: 89.2303x; 1.0471x over previous
import functools

import jax
import jax.numpy as jnp
from jax import lax
from jax.experimental import pallas as pl
from jax.experimental.pallas import tpu as pltpu
from jax.experimental.pallas import tpu_sc as plsc

N_ATOMS = 1048576
N_TYPES = 119
N_PROPS = 64
N_SYSTEMS = 16384

TPAD = 120
NBINS = TPAD * N_SYSTEMS
NC = 2
NS = 16
NW = NC * NS
A_PER_W = N_ATOMS // NW
CHUNK = 1024
NBUF = 6
NCH = A_PER_W // CHUNK
ACC_PER_TILE = NBINS // NS


def _sc_histogram(keys, zeros_h, ones_h):
    mesh = plsc.VectorSubcoreMesh(
        core_axis_name="c", subcore_axis_name="s", num_cores=NC, num_subcores=NS
    )

    @functools.partial(
        pl.kernel,
        mesh=mesh,
        out_type=(
            jax.ShapeDtypeStruct((TPAD, N_SYSTEMS), jnp.float32),
            jax.ShapeDtypeStruct((TPAD, N_SYSTEMS), jnp.float32),
        ),
        scratch_types=[
            [pltpu.VMEM((CHUNK,), jnp.int32) for _ in range(NBUF)],
            pltpu.VMEM((CHUNK,), jnp.float32),
            pltpu.VMEM_SHARED((NBINS,), jnp.float32),
            pltpu.SemaphoreType.DMA,
            [pltpu.SemaphoreType.DMA for _ in range(NBUF)],
            [pltpu.SemaphoreType.DMA for _ in range(NBUF)],
        ],
    )
    def hist(keys_hbm, zeros_hbm, ones_hbm, out0, out1,
             kbufs, ones_v, acc, sem_z, sem_l, sem_s):
        c = lax.axis_index("c")
        s = lax.axis_index("s")
        w = c * NS + s

        zd = pltpu.async_copy(
            zeros_hbm, acc.at[pl.ds(s * ACC_PER_TILE, ACC_PER_TILE)], sem_z)
        pltpu.sync_copy(ones_hbm, ones_v)

        def start_load(k):
            b = k % NBUF
            base = w * A_PER_W + k * CHUNK
            return pltpu.async_copy(keys_hbm.at[pl.ds(base, CHUNK)],
                                    kbufs[b], sem_l[b])

        loads = [start_load(k) for k in range(NBUF - 1)] + [None] * (
            NCH - (NBUF - 1))
        zd.wait()
        plsc.subcore_barrier()

        scatters = [None] * NBUF
        for k in range(NCH):
            b = k % NBUF
            loads[k].wait()
            scatters[b] = pltpu.async_copy(
                ones_v, acc.at[kbufs[b]], sem_s[b], add=True)
            nxt = k + NBUF - 1
            if nxt < NCH:
                nb = nxt % NBUF
                if scatters[nb] is not None:
                    scatters[nb].wait()
                    scatters[nb] = None
                loads[nxt] = start_load(nxt)

        for d in scatters:
            if d is not None:
                d.wait()
        plsc.subcore_barrier()

        for i in range(8):
            r = i * NS + s

            @pl.when(r < TPAD)
            def _(r=r):
                row = acc.at[pl.ds(r * N_SYSTEMS, N_SYSTEMS)]

                @pl.when(c == 0)
                def _():
                    pltpu.sync_copy(row, out0.at[r])

                @pl.when(c == 1)
                def _():
                    pltpu.sync_copy(row, out1.at[r])

    return hist(keys, zeros_h, ones_h)


_BN = 4096
_NB = N_SYSTEMS // _BN


def _fused_body(c0_hbm, c1_hbm, w_ref, o_ref, lhs0, lhs1, sems):
    def start(j, b):
        sl = pl.ds(j * _BN, _BN)
        d0 = pltpu.make_async_copy(c0_hbm.at[:, sl], lhs0.at[b], sems.at[b, 0])
        d1 = pltpu.make_async_copy(c1_hbm.at[:, sl], lhs1.at[b], sems.at[b, 1])
        d0.start()
        d1.start()
        return d0, d1

    pend = start(0, 0)
    for j in range(_NB):
        b = j % 2
        pend[0].wait()
        pend[1].wait()
        if j + 1 < _NB:
            pend = start(j + 1, 1 - b)
        cnt = lhs0[b] + lhs1[b]
        o_ref[pl.ds(j * _BN, _BN), :] = lax.dot_general(
            cnt, w_ref[...], (((0,), (0,)), ((), ())),
            preferred_element_type=jnp.float32,
        )


def _tc_matmul(c0, c1, w_pad):
    return pl.pallas_call(
        _fused_body,
        in_specs=[
            pl.BlockSpec(memory_space=pltpu.HBM),
            pl.BlockSpec(memory_space=pltpu.HBM),
            pl.BlockSpec(memory_space=pltpu.VMEM),
        ],
        out_specs=pl.BlockSpec(memory_space=pltpu.VMEM),
        out_shape=jax.ShapeDtypeStruct((N_SYSTEMS, N_PROPS), jnp.float32),
        scratch_shapes=[
            pltpu.VMEM((2, TPAD, _BN), jnp.float32),
            pltpu.VMEM((2, TPAD, _BN), jnp.float32),
            pltpu.SemaphoreType.DMA((2, 2)),
        ],
    )(c0, c1, w_pad)


def kernel(weights, atom_types, system_ids, n_systems):
    del n_systems
    keys = atom_types * N_SYSTEMS + system_ids
    zeros_h = jnp.zeros((ACC_PER_TILE,), jnp.float32)
    ones_h = jnp.ones((CHUNK,), jnp.float32)
    c0, c1 = _sc_histogram(keys, zeros_h, ones_h)
    w_pad = jnp.zeros((TPAD, N_PROPS), jnp.float32).at[:N_TYPES].set(weights)
    return _tc_matmul(c0, c1, w_pad)


# speedup vs baseline: 90.4280x; 1.0134x over previous
import functools

import jax
import jax.numpy as jnp
from jax import lax
from jax.experimental import pallas as pl
from jax.experimental.pallas import tpu as pltpu
from jax.experimental.pallas import tpu_sc as plsc

N_ATOMS = 1048576
N_TYPES = 119
N_PROPS = 64
N_SYSTEMS = 16384

TPAD = 120
NBINS = TPAD * N_SYSTEMS
NC = 2
NS = 16
NW = NC * NS
A_PER_W = N_ATOMS // NW
CHUNK = 1024
NBUF = 6
NCH = A_PER_W // CHUNK
ACC_PER_TILE = NBINS // NS


def _sc_histogram(keys, zeros_h, ones_h):
    mesh = plsc.VectorSubcoreMesh(
        core_axis_name="c", subcore_axis_name="s", num_cores=NC, num_subcores=NS
    )

    @functools.partial(
        pl.kernel,
        mesh=mesh,
        out_type=(
            jax.ShapeDtypeStruct((TPAD, N_SYSTEMS), jnp.float32),
            jax.ShapeDtypeStruct((TPAD, N_SYSTEMS), jnp.float32),
        ),
        scratch_types=[
            [pltpu.VMEM((CHUNK,), jnp.int32) for _ in range(NBUF)],
            pltpu.VMEM((CHUNK,), jnp.float32),
            pltpu.VMEM_SHARED((NBINS,), jnp.float32),
            pltpu.SemaphoreType.DMA,
            [pltpu.SemaphoreType.DMA for _ in range(NBUF)],
            [pltpu.SemaphoreType.DMA for _ in range(NBUF)],
        ],
    )
    def hist(keys_hbm, zeros_hbm, ones_hbm, out0, out1,
             kbufs, ones_v, acc, sem_z, sem_l, sem_s):
        c = lax.axis_index("c")
        s = lax.axis_index("s")
        w = c * NS + s

        zd = pltpu.async_copy(
            zeros_hbm, acc.at[pl.ds(s * ACC_PER_TILE, ACC_PER_TILE)], sem_z)
        pltpu.sync_copy(ones_hbm, ones_v)

        def start_load(k):
            b = k % NBUF
            base = w * A_PER_W + k * CHUNK
            return pltpu.async_copy(keys_hbm.at[pl.ds(base, CHUNK)],
                                    kbufs[b], sem_l[b])

        loads = [start_load(k) for k in range(NBUF - 1)] + [None] * (
            NCH - (NBUF - 1))
        zd.wait()
        plsc.subcore_barrier()

        scatters = [None] * NBUF
        for k in range(NCH):
            b = k % NBUF
            loads[k].wait()
            scatters[b] = pltpu.async_copy(
                ones_v, acc.at[kbufs[b]], sem_s[b], add=True)
            nxt = k + NBUF - 1
            if nxt < NCH:
                nb = nxt % NBUF
                if scatters[nb] is not None:
                    scatters[nb].wait()
                    scatters[nb] = None
                loads[nxt] = start_load(nxt)

        for d in scatters:
            if d is not None:
                d.wait()
        plsc.subcore_barrier()

        for i in range(8):
            r = i * NS + s

            @pl.when(r < TPAD)
            def _(r=r):
                row = acc.at[pl.ds(r * N_SYSTEMS, N_SYSTEMS)]

                @pl.when(c == 0)
                def _():
                    pltpu.sync_copy(row, out0.at[r])

                @pl.when(c == 1)
                def _():
                    pltpu.sync_copy(row, out1.at[r])

    return hist(keys, zeros_h, ones_h)


_BN = 8192
_NB = N_SYSTEMS // _BN


def _fused_body(c0_hbm, c1_hbm, w_ref, o_ref, lhs0, lhs1, sems):
    def start(j, b):
        sl = pl.ds(j * _BN, _BN)
        d0 = pltpu.make_async_copy(c0_hbm.at[:, sl], lhs0.at[b], sems.at[b, 0])
        d1 = pltpu.make_async_copy(c1_hbm.at[:, sl], lhs1.at[b], sems.at[b, 1])
        d0.start()
        d1.start()
        return d0, d1

    pend = start(0, 0)
    for j in range(_NB):
        b = j % 2
        pend[0].wait()
        pend[1].wait()
        if j + 1 < _NB:
            pend = start(j + 1, 1 - b)
        cnt = lhs0[b] + lhs1[b]
        o_ref[pl.ds(j * _BN, _BN), :] = lax.dot_general(
            cnt, w_ref[...], (((0,), (0,)), ((), ())),
            preferred_element_type=jnp.float32,
        )


def _tc_matmul(c0, c1, w_pad):
    return pl.pallas_call(
        _fused_body,
        in_specs=[
            pl.BlockSpec(memory_space=pltpu.HBM),
            pl.BlockSpec(memory_space=pltpu.HBM),
            pl.BlockSpec(memory_space=pltpu.VMEM),
        ],
        out_specs=pl.BlockSpec(memory_space=pltpu.VMEM),
        out_shape=jax.ShapeDtypeStruct((N_SYSTEMS, N_PROPS), jnp.float32),
        scratch_shapes=[
            pltpu.VMEM((2, TPAD, _BN), jnp.float32),
            pltpu.VMEM((2, TPAD, _BN), jnp.float32),
            pltpu.SemaphoreType.DMA((2, 2)),
        ],
    )(c0, c1, w_pad)


def kernel(weights, atom_types, system_ids, n_systems):
    del n_systems
    keys = atom_types * N_SYSTEMS + system_ids
    zeros_h = jnp.zeros((ACC_PER_TILE,), jnp.float32)
    ones_h = jnp.ones((CHUNK,), jnp.float32)
    c0, c1 = _sc_histogram(keys, zeros_h, ones_h)
    w_pad = jnp.zeros((TPAD, N_PROPS), jnp.float32).at[:N_TYPES].set(weights)
    return _tc_matmul(c0, c1, w_pad)
